# Initial kernel scaffold; baseline (speedup 1.0000x reference)
import jax
import jax.numpy as jnp
from jax.experimental import pallas as pl


def kernel(x, edge_index, Wl1, Wr1, b1, Wl2, Wr2, b2, W1, bl1, W2, bl2):
    raise NotImplementedError("write your pallas kernel here")


# speedup vs baseline: 4.8850x; 4.8850x over previous
import functools

import jax
import jax.numpy as jnp
from jax import lax
from jax.experimental import pallas as pl
from jax.experimental.pallas import tpu as pltpu
from jax.experimental.pallas import tpu_sc as plsc

N = 10000
E = 320000
D = 128
DA = 144
C = 64

NC = 2
NS = 16
NW = NC * NS
EPW = E // NW
CHUNK = 80
NCHUNK = EPW // CHUNK
RPT = N // NS
RBLK = 2000


def _seg_sum_build():
    mesh = plsc.VectorSubcoreMesh(core_axis_name="c", subcore_axis_name="s")

    @functools.partial(
        pl.kernel,
        mesh=mesh,
        out_type=jax.ShapeDtypeStruct((NC * N, DA), jnp.float32),
        compiler_params=pltpu.CompilerParams(use_tc_tiling_on_sc=False),
        scratch_types=[
            pltpu.VMEM_SHARED((N, DA), jnp.float32),
            pltpu.VMEM((CHUNK,), jnp.int32),
            pltpu.VMEM((CHUNK,), jnp.int32),
            pltpu.VMEM((CHUNK, DA), jnp.float32),
            pltpu.SemaphoreType.DMA,
        ],
    )
    def seg_sum(src_hbm, dst_hbm, table_hbm, zeros_hbm, out_hbm,
                acc_sh, sidx_v, didx_v, rows_v, sem):
        c = lax.axis_index("c")
        s = lax.axis_index("s")
        wid = s * NC + c

        r0 = s * RPT
        pltpu.sync_copy(zeros_hbm.at[pl.ds(r0, RPT)],
                        acc_sh.at[pl.ds(r0, RPT)])
        plsc.subcore_barrier()

        def body(i, carry):
            base = wid * EPW + i * CHUNK
            pltpu.sync_copy(src_hbm.at[pl.ds(base, CHUNK)], sidx_v)
            pltpu.sync_copy(dst_hbm.at[pl.ds(base, CHUNK)], didx_v)
            pltpu.async_copy(table_hbm.at[sidx_v], rows_v, sem).wait()
            pltpu.sync_copy(rows_v, acc_sh.at[didx_v], add=True)
            return carry

        lax.fori_loop(0, NCHUNK, body, 0)

        plsc.subcore_barrier()
        pltpu.sync_copy(acc_sh.at[pl.ds(r0, RPT)],
                        out_hbm.at[pl.ds(c * N + r0, RPT)])

    return seg_sum


_seg_sum = _seg_sum_build()


def _sage_block(p_ref, h_ref, wl_ref, wr_ref, b_ref):
    acc = p_ref[0] + p_ref[1]
    agg = acc[:, :D]
    deg16 = jnp.sum(acc[:, D:], axis=1, keepdims=True)
    mean = agg * (16.0 / jnp.maximum(deg16, 16.0))
    h = jnp.dot(mean, wl_ref[...], preferred_element_type=jnp.float32)
    h = h + b_ref[...]
    h = h + jnp.dot(h_ref[...][:, :D], wr_ref[...],
                    preferred_element_type=jnp.float32)
    return jnp.maximum(h, 0.0)


def _layer1_body(p_ref, x_ref, wl_ref, wr_ref, b_ref, o_ref):
    h = _sage_block(p_ref, x_ref, wl_ref, wr_ref, b_ref)
    o_ref[...] = jnp.concatenate(
        [h, jnp.ones((RBLK, DA - D), jnp.float32)], axis=1)


def _layer2_body(p_ref, h_ref, wl_ref, wr_ref, b_ref,
                 w1_ref, bl1_ref, w2_ref, bl2_ref, o_ref):
    h = _sage_block(p_ref, h_ref, wl_ref, wr_ref, b_ref)
    h = jnp.maximum(
        jnp.dot(h, w1_ref[...], preferred_element_type=jnp.float32)
        + bl1_ref[...], 0.0)
    o_ref[...] = (jnp.dot(h, w2_ref[...], preferred_element_type=jnp.float32)
                  + bl2_ref[...])


_mat_spec = pl.BlockSpec((D, D), lambda i: (0, 0))
_vec_spec = pl.BlockSpec((1, D), lambda i: (0, 0))
_p_spec = pl.BlockSpec((NC, RBLK, DA), lambda i: (0, i, 0))
_haug_spec = pl.BlockSpec((RBLK, DA), lambda i: (i, 0))

_layer1 = pl.pallas_call(
    _layer1_body,
    grid=(N // RBLK,),
    in_specs=[_p_spec,
              _haug_spec,
              _mat_spec, _mat_spec, _vec_spec],
    out_specs=_haug_spec,
    out_shape=jax.ShapeDtypeStruct((N, DA), jnp.float32),
)

_layer2 = pl.pallas_call(
    _layer2_body,
    grid=(N // RBLK,),
    in_specs=[_p_spec,
              _haug_spec,
              _mat_spec, _mat_spec, _vec_spec,
              _mat_spec, _vec_spec,
              pl.BlockSpec((D, C), lambda i: (0, 0)),
              pl.BlockSpec((1, C), lambda i: (0, 0))],
    out_specs=pl.BlockSpec((RBLK, C), lambda i: (i, 0)),
    out_shape=jax.ShapeDtypeStruct((N, C), jnp.float32),
)


def kernel(x, edge_index, Wl1, Wr1, b1, Wl2, Wr2, b2, W1, bl1, W2, bl2):
    src = edge_index[0]
    dst = edge_index[1]
    xaug = jnp.concatenate([x, jnp.ones((N, DA - D), jnp.float32)], axis=1)
    zeros = jnp.zeros((N, DA), jnp.float32)

    p1 = _seg_sum(src, dst, xaug, zeros).reshape(NC, N, DA)
    h1 = _layer1(p1, xaug, Wl1, Wr1, b1.reshape(1, D))
    p2 = _seg_sum(src, dst, h1, zeros).reshape(NC, N, DA)
    out = _layer2(p2, h1, Wl2, Wr2, b2.reshape(1, D),
                  W1, bl1.reshape(1, D), W2, bl2.reshape(1, C))
    return out


# speedup vs baseline: 6.2224x; 1.2738x over previous
import functools

import jax
import jax.numpy as jnp
from jax import lax
from jax.experimental import pallas as pl
from jax.experimental.pallas import tpu as pltpu
from jax.experimental.pallas import tpu_sc as plsc

N = 10000
E = 320000
D = 128
DA = 144
C = 64

NC = 2
NS = 16
NW = NC * NS
EPW = E // NW
CHUNK = 40
NCHUNK = EPW // CHUNK
RPT = N // NS
RBLK = 2000


def _seg_sum_build():
    mesh = plsc.VectorSubcoreMesh(core_axis_name="c", subcore_axis_name="s")

    @functools.partial(
        pl.kernel,
        mesh=mesh,
        out_type=jax.ShapeDtypeStruct((NC * N, DA), jnp.float32),
        compiler_params=pltpu.CompilerParams(use_tc_tiling_on_sc=False),
        scratch_types=[
            pltpu.VMEM_SHARED((N, DA), jnp.float32),
            pltpu.VMEM((NCHUNK, CHUNK), jnp.int32),
            pltpu.VMEM((NCHUNK, CHUNK), jnp.int32),
            pltpu.VMEM((CHUNK, DA), jnp.float32),
            pltpu.VMEM((CHUNK, DA), jnp.float32),
            pltpu.SemaphoreType.DMA,
            pltpu.SemaphoreType.DMA,
            pltpu.SemaphoreType.DMA,
            pltpu.SemaphoreType.DMA,
        ],
    )
    def seg_sum(src_hbm, dst_hbm, table_hbm, zeros_hbm, out_hbm,
                acc_sh, sidx_v, didx_v, rows0_v, rows1_v,
                sem_g0, sem_g1, sem_s0, sem_s1):
        c = lax.axis_index("c")
        s = lax.axis_index("s")
        wid = s * NC + c

        r0 = s * RPT
        pltpu.sync_copy(zeros_hbm.at[pl.ds(r0, RPT)],
                        acc_sh.at[pl.ds(r0, RPT)])
        pltpu.sync_copy(src_hbm.at[wid], sidx_v)
        pltpu.sync_copy(dst_hbm.at[wid], didx_v)
        plsc.subcore_barrier()

        def gather(i, rows_v, sem):
            pltpu.async_copy(table_hbm.at[sidx_v.at[i]], rows_v, sem)

        def gather_wait(i, rows_v, sem):
            pltpu.make_async_copy(table_hbm.at[sidx_v.at[i]], rows_v,
                                  sem).wait()

        def scatter(i, rows_v, sem):
            pltpu.async_copy(rows_v, acc_sh.at[didx_v.at[i]], sem, add=True)

        def scatter_wait(i, rows_v, sem):
            pltpu.make_async_copy(rows_v, acc_sh.at[didx_v.at[i]],
                                  sem).wait()

        gather(0, rows0_v, sem_g0)
        gather_wait(0, rows0_v, sem_g0)
        scatter(0, rows0_v, sem_s0)
        gather(1, rows1_v, sem_g1)

        def pair(p, carry):
            i0 = 2 * p + 1
            gather_wait(i0, rows1_v, sem_g1)
            scatter(i0, rows1_v, sem_s1)
            scatter_wait(i0 - 1, rows0_v, sem_s0)
            gather(i0 + 1, rows0_v, sem_g0)
            gather_wait(i0 + 1, rows0_v, sem_g0)
            scatter(i0 + 1, rows0_v, sem_s0)
            scatter_wait(i0, rows1_v, sem_s1)
            gather(i0 + 2, rows1_v, sem_g1)
            return carry

        lax.fori_loop(0, (NCHUNK - 2) // 2, pair, 0)

        last = NCHUNK - 1
        gather_wait(last, rows1_v, sem_g1)
        scatter(last, rows1_v, sem_s1)
        scatter_wait(last - 1, rows0_v, sem_s0)
        scatter_wait(last, rows1_v, sem_s1)

        plsc.subcore_barrier()
        pltpu.sync_copy(acc_sh.at[pl.ds(r0, RPT)],
                        out_hbm.at[pl.ds(c * N + r0, RPT)])

    return seg_sum


_seg_sum = _seg_sum_build()


def _sage_block(p_ref, h_ref, wl_ref, wr_ref, b_ref):
    acc = p_ref[0] + p_ref[1]
    agg = acc[:, :D]
    deg16 = jnp.sum(acc[:, D:], axis=1, keepdims=True)
    mean = agg * (16.0 / jnp.maximum(deg16, 16.0))
    h = jnp.dot(mean, wl_ref[...], preferred_element_type=jnp.float32)
    h = h + b_ref[...]
    h = h + jnp.dot(h_ref[...][:, :D], wr_ref[...],
                    preferred_element_type=jnp.float32)
    return jnp.maximum(h, 0.0)


def _layer1_body(p_ref, x_ref, wl_ref, wr_ref, b_ref, o_ref):
    h = _sage_block(p_ref, x_ref, wl_ref, wr_ref, b_ref)
    o_ref[...] = jnp.concatenate(
        [h, jnp.ones((RBLK, DA - D), jnp.float32)], axis=1)


def _layer2_body(p_ref, h_ref, wl_ref, wr_ref, b_ref,
                 w1_ref, bl1_ref, w2_ref, bl2_ref, o_ref):
    h = _sage_block(p_ref, h_ref, wl_ref, wr_ref, b_ref)
    h = jnp.maximum(
        jnp.dot(h, w1_ref[...], preferred_element_type=jnp.float32)
        + bl1_ref[...], 0.0)
    o_ref[...] = (jnp.dot(h, w2_ref[...], preferred_element_type=jnp.float32)
                  + bl2_ref[...])


_mat_spec = pl.BlockSpec((D, D), lambda i: (0, 0))
_vec_spec = pl.BlockSpec((1, D), lambda i: (0, 0))
_p_spec = pl.BlockSpec((NC, RBLK, DA), lambda i: (0, i, 0))
_haug_spec = pl.BlockSpec((RBLK, DA), lambda i: (i, 0))

_layer1 = pl.pallas_call(
    _layer1_body,
    grid=(N // RBLK,),
    in_specs=[_p_spec,
              _haug_spec,
              _mat_spec, _mat_spec, _vec_spec],
    out_specs=_haug_spec,
    out_shape=jax.ShapeDtypeStruct((N, DA), jnp.float32),
)

_layer2 = pl.pallas_call(
    _layer2_body,
    grid=(N // RBLK,),
    in_specs=[_p_spec,
              _haug_spec,
              _mat_spec, _mat_spec, _vec_spec,
              _mat_spec, _vec_spec,
              pl.BlockSpec((D, C), lambda i: (0, 0)),
              pl.BlockSpec((1, C), lambda i: (0, 0))],
    out_specs=pl.BlockSpec((RBLK, C), lambda i: (i, 0)),
    out_shape=jax.ShapeDtypeStruct((N, C), jnp.float32),
)


def kernel(x, edge_index, Wl1, Wr1, b1, Wl2, Wr2, b2, W1, bl1, W2, bl2):
    src = edge_index[0].reshape(NW, NCHUNK, CHUNK)
    dst = edge_index[1].reshape(NW, NCHUNK, CHUNK)
    xaug = jnp.concatenate([x, jnp.ones((N, DA - D), jnp.float32)], axis=1)
    zeros = jnp.zeros((N, DA), jnp.float32)

    p1 = _seg_sum(src, dst, xaug, zeros).reshape(NC, N, DA)
    h1 = _layer1(p1, xaug, Wl1, Wr1, b1.reshape(1, D))
    p2 = _seg_sum(src, dst, h1, zeros).reshape(NC, N, DA)
    out = _layer2(p2, h1, Wl2, Wr2, b2.reshape(1, D),
                  W1, bl1.reshape(1, D), W2, bl2.reshape(1, C))
    return out


# speedup vs baseline: 7.6810x; 1.2344x over previous
import functools

import jax
import jax.numpy as jnp
from jax import lax
from jax.experimental import pallas as pl
from jax.experimental.pallas import tpu as pltpu
from jax.experimental.pallas import tpu_sc as plsc

N = 10000
E = 320000
D = 128
DA = 144
C = 64

NC = 2
NS = 16
NW = NC * NS
EPW = E // NW
CH1 = 40
CH2 = 80
RPT = N // NS
RBLK = 2000


def _seg_sum_build(feat, chunk):
    nchunk = EPW // chunk
    mesh = plsc.VectorSubcoreMesh(core_axis_name="c", subcore_axis_name="s")

    @functools.partial(
        pl.kernel,
        mesh=mesh,
        out_type=jax.ShapeDtypeStruct((NC * N, feat), jnp.float32),
        compiler_params=pltpu.CompilerParams(use_tc_tiling_on_sc=False),
        scratch_types=[
            pltpu.VMEM_SHARED((N, feat), jnp.float32),
            pltpu.VMEM((nchunk, chunk), jnp.int32),
            pltpu.VMEM((nchunk, chunk), jnp.int32),
            pltpu.VMEM((chunk, feat), jnp.float32),
            pltpu.VMEM((chunk, feat), jnp.float32),
            pltpu.SemaphoreType.DMA,
            pltpu.SemaphoreType.DMA,
            pltpu.SemaphoreType.DMA,
            pltpu.SemaphoreType.DMA,
        ],
    )
    def seg_sum(src_hbm, dst_hbm, table_hbm, zeros_hbm, out_hbm,
                acc_sh, sidx_v, didx_v, rows0_v, rows1_v,
                sem_g0, sem_g1, sem_s0, sem_s1):
        c = lax.axis_index("c")
        s = lax.axis_index("s")
        wid = s * NC + c

        r0 = s * RPT
        pltpu.sync_copy(zeros_hbm.at[pl.ds(r0, RPT)],
                        acc_sh.at[pl.ds(r0, RPT)])
        pltpu.sync_copy(src_hbm.at[wid], sidx_v)
        pltpu.sync_copy(dst_hbm.at[wid], didx_v)
        plsc.subcore_barrier()

        def gather(i, rows_v, sem):
            pltpu.async_copy(table_hbm.at[sidx_v.at[i]], rows_v, sem)

        def gather_wait(i, rows_v, sem):
            pltpu.make_async_copy(table_hbm.at[sidx_v.at[i]], rows_v,
                                  sem).wait()

        def scatter(i, rows_v, sem):
            pltpu.async_copy(rows_v, acc_sh.at[didx_v.at[i]], sem, add=True)

        def scatter_wait(i, rows_v, sem):
            pltpu.make_async_copy(rows_v, acc_sh.at[didx_v.at[i]],
                                  sem).wait()

        gather(0, rows0_v, sem_g0)
        gather_wait(0, rows0_v, sem_g0)
        scatter(0, rows0_v, sem_s0)
        gather(1, rows1_v, sem_g1)

        def pair(p, carry):
            i0 = 2 * p + 1
            gather_wait(i0, rows1_v, sem_g1)
            scatter(i0, rows1_v, sem_s1)
            scatter_wait(i0 - 1, rows0_v, sem_s0)
            gather(i0 + 1, rows0_v, sem_g0)
            gather_wait(i0 + 1, rows0_v, sem_g0)
            scatter(i0 + 1, rows0_v, sem_s0)
            scatter_wait(i0, rows1_v, sem_s1)
            gather(i0 + 2, rows1_v, sem_g1)
            return carry

        lax.fori_loop(0, (nchunk - 2) // 2, pair, 0)

        last = nchunk - 1
        if nchunk % 2 == 0:
            gather_wait(last, rows1_v, sem_g1)
            scatter(last, rows1_v, sem_s1)
            scatter_wait(last - 1, rows0_v, sem_s0)
            scatter_wait(last, rows1_v, sem_s1)
        else:
            gather_wait(last - 1, rows1_v, sem_g1)
            scatter(last - 1, rows1_v, sem_s1)
            scatter_wait(last - 2, rows0_v, sem_s0)
            gather(last, rows0_v, sem_g0)
            gather_wait(last, rows0_v, sem_g0)
            scatter(last, rows0_v, sem_s0)
            scatter_wait(last - 1, rows1_v, sem_s1)
            scatter_wait(last, rows0_v, sem_s0)

        plsc.subcore_barrier()
        pltpu.sync_copy(acc_sh.at[pl.ds(r0, RPT)],
                        out_hbm.at[pl.ds(c * N + r0, RPT)])

    return seg_sum


_seg_sum1 = _seg_sum_build(DA, CH1)
_seg_sum2 = _seg_sum_build(D, CH2)


def _layer1_body(p_ref, x_ref, wl_ref, wr_ref, b_ref, o_ref, deg_ref):
    acc = p_ref[0] + p_ref[1]
    agg = acc[:, :D]
    deg16 = jnp.sum(acc[:, D:], axis=1, keepdims=True)
    mean = agg * (16.0 / jnp.maximum(deg16, 16.0))
    h = jnp.dot(mean, wl_ref[...], preferred_element_type=jnp.float32)
    h = h + b_ref[...]
    h = h + jnp.dot(x_ref[...][:, :D], wr_ref[...],
                    preferred_element_type=jnp.float32)
    o_ref[...] = jnp.maximum(h, 0.0)
    deg_ref[...] = jnp.broadcast_to(deg16 * (1.0 / 16.0), (RBLK, 8))


def _layer2_body(p_ref, h_ref, deg_ref, wl_ref, wr_ref, b_ref,
                 w1_ref, bl1_ref, w2_ref, bl2_ref, o_ref):
    agg = p_ref[0] + p_ref[1]
    deg = deg_ref[...][:, :1]
    mean = agg * (1.0 / jnp.maximum(deg, 1.0))
    h = jnp.dot(mean, wl_ref[...], preferred_element_type=jnp.float32)
    h = h + b_ref[...]
    h = h + jnp.dot(h_ref[...], wr_ref[...],
                    preferred_element_type=jnp.float32)
    h = jnp.maximum(h, 0.0)
    h = jnp.maximum(
        jnp.dot(h, w1_ref[...], preferred_element_type=jnp.float32)
        + bl1_ref[...], 0.0)
    o_ref[...] = (jnp.dot(h, w2_ref[...], preferred_element_type=jnp.float32)
                  + bl2_ref[...])


_mat_spec = pl.BlockSpec((D, D), lambda i: (0, 0))
_vec_spec = pl.BlockSpec((1, D), lambda i: (0, 0))
_h_spec = pl.BlockSpec((RBLK, D), lambda i: (i, 0))
_deg_spec = pl.BlockSpec((RBLK, 8), lambda i: (i, 0))

_layer1 = pl.pallas_call(
    _layer1_body,
    grid=(N // RBLK,),
    in_specs=[pl.BlockSpec((NC, RBLK, DA), lambda i: (0, i, 0)),
              pl.BlockSpec((RBLK, DA), lambda i: (i, 0)),
              _mat_spec, _mat_spec, _vec_spec],
    out_specs=[_h_spec, _deg_spec],
    out_shape=[jax.ShapeDtypeStruct((N, D), jnp.float32),
               jax.ShapeDtypeStruct((N, 8), jnp.float32)],
)

_layer2 = pl.pallas_call(
    _layer2_body,
    grid=(N // RBLK,),
    in_specs=[pl.BlockSpec((NC, RBLK, D), lambda i: (0, i, 0)),
              _h_spec, _deg_spec,
              _mat_spec, _mat_spec, _vec_spec,
              _mat_spec, _vec_spec,
              pl.BlockSpec((D, C), lambda i: (0, 0)),
              pl.BlockSpec((1, C), lambda i: (0, 0))],
    out_specs=pl.BlockSpec((RBLK, C), lambda i: (i, 0)),
    out_shape=jax.ShapeDtypeStruct((N, C), jnp.float32),
)


def kernel(x, edge_index, Wl1, Wr1, b1, Wl2, Wr2, b2, W1, bl1, W2, bl2):
    src1 = edge_index[0].reshape(NW, EPW // CH1, CH1)
    dst1 = edge_index[1].reshape(NW, EPW // CH1, CH1)
    src2 = edge_index[0].reshape(NW, EPW // CH2, CH2)
    dst2 = edge_index[1].reshape(NW, EPW // CH2, CH2)
    xaug = jnp.concatenate([x, jnp.ones((N, DA - D), jnp.float32)], axis=1)
    zeros1 = jnp.zeros((N, DA), jnp.float32)
    zeros2 = jnp.zeros((N, D), jnp.float32)

    p1 = _seg_sum1(src1, dst1, xaug, zeros1).reshape(NC, N, DA)
    h1, deg = _layer1(p1, xaug, Wl1, Wr1, b1.reshape(1, D))
    p2 = _seg_sum2(src2, dst2, h1, zeros2).reshape(NC, N, D)
    out = _layer2(p2, h1, deg, Wl2, Wr2, b2.reshape(1, D),
                  W1, bl1.reshape(1, D), W2, bl2.reshape(1, C))
    return out


# speedup vs baseline: 10.1120x; 1.3165x over previous
import functools

import jax
import jax.numpy as jnp
from jax import lax
from jax.experimental import pallas as pl
from jax.experimental.pallas import tpu as pltpu
from jax.experimental.pallas import tpu_sc as plsc

N = 10000
E = 320000
D = 128
C = 64

NC = 2
NS = 16
NW = NC * NS
EPW = E // NW
CHUNK = 80
NCHUNK = EPW // CHUNK
RPT = N // NS
RBLK = 2000
L = 16


def _seg_sum_build(with_deg):
    mesh = plsc.VectorSubcoreMesh(core_axis_name="c", subcore_axis_name="s")
    out_type = [jax.ShapeDtypeStruct((NC * N, D), jnp.float32)]
    scratch = [
        pltpu.VMEM_SHARED((N, D), jnp.float32),
        pltpu.VMEM((NCHUNK, CHUNK), jnp.int32),
        pltpu.VMEM((NCHUNK, CHUNK), jnp.int32),
        pltpu.VMEM((CHUNK, D), jnp.float32),
        pltpu.VMEM((CHUNK, D), jnp.float32),
        pltpu.SemaphoreType.DMA,
        pltpu.SemaphoreType.DMA,
        pltpu.SemaphoreType.DMA,
        pltpu.SemaphoreType.DMA,
    ]
    if with_deg:
        out_type.append(
            jax.ShapeDtypeStruct((N // RBLK, NW, RBLK), jnp.float32))
        scratch.append(pltpu.VMEM((N,), jnp.float32))

    @functools.partial(
        pl.kernel,
        mesh=mesh,
        out_type=out_type,
        compiler_params=pltpu.CompilerParams(use_tc_tiling_on_sc=False,
                                             needs_layout_passes=False),
        scratch_types=scratch,
    )
    def seg_sum(src_hbm, dst_hbm, table_hbm, zeros_hbm, out_hbm, *rest):
        if with_deg:
            (deg_hbm, acc_sh, sidx_v, didx_v, rows0_v, rows1_v,
             sem_g0, sem_g1, sem_s0, sem_s1, deg_v) = rest
        else:
            (acc_sh, sidx_v, didx_v, rows0_v, rows1_v,
             sem_g0, sem_g1, sem_s0, sem_s1) = rest
        c = lax.axis_index("c")
        s = lax.axis_index("s")
        wid = s * NC + c

        r0 = s * RPT
        pltpu.sync_copy(zeros_hbm.at[pl.ds(r0, RPT)],
                        acc_sh.at[pl.ds(r0, RPT)])
        pltpu.sync_copy(src_hbm.at[wid], sidx_v)
        pltpu.sync_copy(dst_hbm.at[wid], didx_v)
        if with_deg:
            zv = jnp.zeros((L,), jnp.float32)

            def zero_body(i, carry):
                deg_v[pl.ds(i * L, L)] = zv
                return carry
            lax.fori_loop(0, N // L, zero_body, 0)
        plsc.subcore_barrier()

        def gather(i, rows_v, sem):
            pltpu.async_copy(table_hbm.at[sidx_v.at[i]], rows_v, sem)

        def gather_wait(i, rows_v, sem):
            pltpu.make_async_copy(table_hbm.at[sidx_v.at[i]], rows_v,
                                  sem).wait()

        def scatter(i, rows_v, sem):
            pltpu.async_copy(rows_v, acc_sh.at[didx_v.at[i]], sem, add=True)

        def scatter_wait(i, rows_v, sem):
            pltpu.make_async_copy(rows_v, acc_sh.at[didx_v.at[i]],
                                  sem).wait()

        if with_deg:
            ones = jnp.ones((L,), jnp.float32)

            def count(i):
                for j in range(CHUNK // L):
                    dv = didx_v[i, pl.ds(j * L, L)]
                    plsc.addupdate_scatter(deg_v, [dv], ones)
        else:
            def count(i):
                pass

        gather(0, rows0_v, sem_g0)
        gather_wait(0, rows0_v, sem_g0)
        scatter(0, rows0_v, sem_s0)
        gather(1, rows1_v, sem_g1)
        count(0)

        def pair(p, carry):
            i0 = 2 * p + 1
            gather_wait(i0, rows1_v, sem_g1)
            scatter(i0, rows1_v, sem_s1)
            scatter_wait(i0 - 1, rows0_v, sem_s0)
            gather(i0 + 1, rows0_v, sem_g0)
            count(i0)
            gather_wait(i0 + 1, rows0_v, sem_g0)
            scatter(i0 + 1, rows0_v, sem_s0)
            scatter_wait(i0, rows1_v, sem_s1)
            gather(i0 + 2, rows1_v, sem_g1)
            count(i0 + 1)
            return carry

        lax.fori_loop(0, (NCHUNK - 2) // 2, pair, 0)

        last = NCHUNK - 1
        if NCHUNK % 2 == 0:
            gather_wait(last, rows1_v, sem_g1)
            scatter(last, rows1_v, sem_s1)
            count(last)
            scatter_wait(last - 1, rows0_v, sem_s0)
            scatter_wait(last, rows1_v, sem_s1)
        else:
            gather_wait(last - 1, rows1_v, sem_g1)
            scatter(last - 1, rows1_v, sem_s1)
            scatter_wait(last - 2, rows0_v, sem_s0)
            gather(last, rows0_v, sem_g0)
            count(last - 1)
            gather_wait(last, rows0_v, sem_g0)
            scatter(last, rows0_v, sem_s0)
            count(last)
            scatter_wait(last - 1, rows1_v, sem_s1)
            scatter_wait(last, rows0_v, sem_s0)

        plsc.subcore_barrier()
        pltpu.sync_copy(acc_sh.at[pl.ds(r0, RPT)],
                        out_hbm.at[pl.ds(c * N + r0, RPT)])
        if with_deg:
            for k in range(N // RBLK):
                pltpu.sync_copy(deg_v.at[pl.ds(k * RBLK, RBLK)],
                                deg_hbm.at[k].at[wid])

    return seg_sum


_seg_sum1 = _seg_sum_build(True)
_seg_sum2 = _seg_sum_build(False)


def _layer1_body(p_ref, dp_ref, x_ref, wl_ref, wr_ref, b_ref,
                 o_ref, deg_ref):
    agg = p_ref[0] + p_ref[1]
    deg = jnp.sum(dp_ref[0], axis=0).reshape(RBLK, 1)
    mean = agg * (1.0 / jnp.maximum(deg, 1.0))
    h = jnp.dot(mean, wl_ref[...], preferred_element_type=jnp.float32)
    h = h + b_ref[...]
    h = h + jnp.dot(x_ref[...], wr_ref[...],
                    preferred_element_type=jnp.float32)
    o_ref[...] = jnp.maximum(h, 0.0)
    deg_ref[...] = jnp.broadcast_to(deg, (RBLK, 8))


def _layer2_body(p_ref, h_ref, deg_ref, wl_ref, wr_ref, b_ref,
                 w1_ref, bl1_ref, w2_ref, bl2_ref, o_ref):
    agg = p_ref[0] + p_ref[1]
    deg = deg_ref[...][:, :1]
    mean = agg * (1.0 / jnp.maximum(deg, 1.0))
    h = jnp.dot(mean, wl_ref[...], preferred_element_type=jnp.float32)
    h = h + b_ref[...]
    h = h + jnp.dot(h_ref[...], wr_ref[...],
                    preferred_element_type=jnp.float32)
    h = jnp.maximum(h, 0.0)
    h = jnp.maximum(
        jnp.dot(h, w1_ref[...], preferred_element_type=jnp.float32)
        + bl1_ref[...], 0.0)
    o_ref[...] = (jnp.dot(h, w2_ref[...], preferred_element_type=jnp.float32)
                  + bl2_ref[...])


_mat_spec = pl.BlockSpec((D, D), lambda i: (0, 0))
_vec_spec = pl.BlockSpec((1, D), lambda i: (0, 0))
_h_spec = pl.BlockSpec((RBLK, D), lambda i: (i, 0))
_deg_spec = pl.BlockSpec((RBLK, 8), lambda i: (i, 0))
_p_spec = pl.BlockSpec((NC, RBLK, D), lambda i: (0, i, 0))

_layer1 = pl.pallas_call(
    _layer1_body,
    grid=(N // RBLK,),
    in_specs=[_p_spec,
              pl.BlockSpec((1, NW, RBLK), lambda i: (i, 0, 0)),
              _h_spec,
              _mat_spec, _mat_spec, _vec_spec],
    out_specs=[_h_spec, _deg_spec],
    out_shape=[jax.ShapeDtypeStruct((N, D), jnp.float32),
               jax.ShapeDtypeStruct((N, 8), jnp.float32)],
)

_layer2 = pl.pallas_call(
    _layer2_body,
    grid=(N // RBLK,),
    in_specs=[_p_spec,
              _h_spec, _deg_spec,
              _mat_spec, _mat_spec, _vec_spec,
              _mat_spec, _vec_spec,
              pl.BlockSpec((D, C), lambda i: (0, 0)),
              pl.BlockSpec((1, C), lambda i: (0, 0))],
    out_specs=pl.BlockSpec((RBLK, C), lambda i: (i, 0)),
    out_shape=jax.ShapeDtypeStruct((N, C), jnp.float32),
)


def kernel(x, edge_index, Wl1, Wr1, b1, Wl2, Wr2, b2, W1, bl1, W2, bl2):
    src = edge_index[0].reshape(NW, NCHUNK, CHUNK)
    dst = edge_index[1].reshape(NW, NCHUNK, CHUNK)
    zeros = jnp.zeros((N, D), jnp.float32)

    p1, dp = _seg_sum1(src, dst, x, zeros)
    p1 = p1.reshape(NC, N, D)
    h1, deg = _layer1(p1, dp, x, Wl1, Wr1, b1.reshape(1, D))
    p2, = _seg_sum2(src, dst, h1, zeros)
    p2 = p2.reshape(NC, N, D)
    out = _layer2(p2, h1, deg, Wl2, Wr2, b2.reshape(1, D),
                  W1, bl1.reshape(1, D), W2, bl2.reshape(1, C))
    return out
